# Initial kernel scaffold; baseline (speedup 1.0000x reference)
import jax
import jax.numpy as jnp
from jax.experimental import pallas as pl


def kernel(X, A, W, a_prelu, Wfc, perm):
    raise NotImplementedError("write your pallas kernel here")


# speedup vs baseline: 1.3313x; 1.3313x over previous
import jax
import jax.numpy as jnp
from jax.experimental import pallas as pl
from jax.experimental.pallas import tpu as pltpu

N = 10000
F = 128
TM = 200


def _xw_body(x_ref, w_ref, xw_ref):
    xw_ref[...] = jnp.dot(x_ref[...], w_ref[...],
                          preferred_element_type=jnp.float32)


def _matmul_xw(X, W):
    return pl.pallas_call(
        _xw_body,
        grid=(5,),
        in_specs=[
            pl.BlockSpec((2000, F), lambda i: (i, 0)),
            pl.BlockSpec((F, F), lambda i: (0, 0)),
        ],
        out_specs=pl.BlockSpec((2000, F), lambda i: (i, 0)),
        out_shape=jax.ShapeDtypeStruct((N, F), jnp.float32),
    )(X, W)


def _main_body(ap_ref, a_ref, xw_ref, xwp_ref, h_ref, hn_ref, cs_ref):
    a = a_ref[...]
    c1 = jnp.dot(a, xw_ref[...], preferred_element_type=jnp.float32)
    c2 = jnp.dot(a, xwp_ref[...], preferred_element_type=jnp.float32)
    al = ap_ref[0]
    h1 = jnp.where(c1 >= 0, c1, al * c1)
    h2 = jnp.where(c2 >= 0, c2, al * c2)
    h_ref[...] = h1
    hn_ref[...] = h2
    part = jnp.sum(h1, axis=0, keepdims=True)

    @pl.when(pl.program_id(0) == 0)
    def _init():
        cs_ref[...] = part

    @pl.when(pl.program_id(0) > 0)
    def _acc():
        cs_ref[...] += part


def _main(A, XW, XWp, a_prelu):
    return pl.pallas_call(
        _main_body,
        grid=(N // TM,),
        in_specs=[
            pl.BlockSpec(memory_space=pltpu.SMEM),
            pl.BlockSpec((TM, N), lambda i: (i, 0)),
            pl.BlockSpec((N, F), lambda i: (0, 0)),
            pl.BlockSpec((N, F), lambda i: (0, 0)),
        ],
        out_specs=[
            pl.BlockSpec((TM, F), lambda i: (i, 0)),
            pl.BlockSpec((TM, F), lambda i: (i, 0)),
            pl.BlockSpec((1, F), lambda i: (0, 0)),
        ],
        out_shape=[
            jax.ShapeDtypeStruct((N, F), jnp.float32),
            jax.ShapeDtypeStruct((N, F), jnp.float32),
            jax.ShapeDtypeStruct((1, F), jnp.float32),
        ],
    )(a_prelu.reshape(1), A, XW, XWp)


def _readout_body(h_ref, hn_ref, cs_ref, wfc_ref, o1_ref, o2_ref):
    s = jax.nn.sigmoid(cs_ref[...] * (1.0 / N))
    x = jnp.sum(wfc_ref[...] * s, axis=1)
    o1_ref[...] = jnp.sum(h_ref[...] * x[None, :], axis=1)
    o2_ref[...] = jnp.sum(hn_ref[...] * x[None, :], axis=1)


def _readout(H, Hn, cs, Wfc):
    return pl.pallas_call(
        _readout_body,
        grid=(1,),
        in_specs=[
            pl.BlockSpec((N, F), lambda i: (0, 0)),
            pl.BlockSpec((N, F), lambda i: (0, 0)),
            pl.BlockSpec((1, F), lambda i: (0, 0)),
            pl.BlockSpec((F, F), lambda i: (0, 0)),
        ],
        out_specs=[
            pl.BlockSpec((N,), lambda i: (0,)),
            pl.BlockSpec((N,), lambda i: (0,)),
        ],
        out_shape=[
            jax.ShapeDtypeStruct((N,), jnp.float32),
            jax.ShapeDtypeStruct((N,), jnp.float32),
        ],
    )(H, Hn, cs, Wfc)


def kernel(X, A, W, a_prelu, Wfc, perm):
    XW = _matmul_xw(X, W)
    XWp = jnp.take(XW, perm, axis=0)
    H, Hn, cs = _main(A, XW, XWp, a_prelu)
    o1, o2 = _readout(H, Hn, cs, Wfc)
    out = jnp.concatenate([o1, o2], axis=0)
    labels = jnp.concatenate([
        jnp.ones((N,), dtype=jnp.float32),
        jnp.zeros((N,), dtype=jnp.float32),
    ])
    return (out, labels, jnp.array(0.0, dtype=jnp.float32))


# speedup vs baseline: 1.3874x; 1.0422x over previous
import jax
import jax.numpy as jnp
from jax.experimental import pallas as pl
from jax.experimental.pallas import tpu as pltpu

N = 10000
F = 128
TM = 200


def _xw_body(x_ref, w_ref, xw_ref):
    xw_ref[...] = jnp.dot(x_ref[...], w_ref[...],
                          preferred_element_type=jnp.float32
                          ).astype(jnp.bfloat16)


def _matmul_xw(X, W):
    return pl.pallas_call(
        _xw_body,
        grid=(5,),
        in_specs=[
            pl.BlockSpec((2000, F), lambda i: (i, 0)),
            pl.BlockSpec((F, F), lambda i: (0, 0)),
        ],
        out_specs=pl.BlockSpec((2000, F), lambda i: (i, 0)),
        out_shape=jax.ShapeDtypeStruct((N, F), jnp.bfloat16),
    )(X, W)


def _main_body(ap_ref, a_ref, xw_ref, xwp_ref, h_ref, hn_ref, cs_ref):
    a = a_ref[...].astype(jnp.bfloat16)
    c1 = jnp.dot(a, xw_ref[...], preferred_element_type=jnp.float32)
    c2 = jnp.dot(a, xwp_ref[...], preferred_element_type=jnp.float32)
    al = ap_ref[0]
    h1 = jnp.where(c1 >= 0, c1, al * c1)
    h2 = jnp.where(c2 >= 0, c2, al * c2)
    h_ref[...] = h1
    hn_ref[...] = h2
    part = jnp.sum(h1, axis=0, keepdims=True)

    @pl.when(pl.program_id(0) == 0)
    def _init():
        cs_ref[...] = part

    @pl.when(pl.program_id(0) > 0)
    def _acc():
        cs_ref[...] += part


def _main(A, XW, XWp, a_prelu):
    return pl.pallas_call(
        _main_body,
        grid=(N // TM,),
        in_specs=[
            pl.BlockSpec(memory_space=pltpu.SMEM),
            pl.BlockSpec((TM, N), lambda i: (i, 0)),
            pl.BlockSpec((N, F), lambda i: (0, 0)),
            pl.BlockSpec((N, F), lambda i: (0, 0)),
        ],
        out_specs=[
            pl.BlockSpec((TM, F), lambda i: (i, 0)),
            pl.BlockSpec((TM, F), lambda i: (i, 0)),
            pl.BlockSpec((1, F), lambda i: (0, 0)),
        ],
        out_shape=[
            jax.ShapeDtypeStruct((N, F), jnp.float32),
            jax.ShapeDtypeStruct((N, F), jnp.float32),
            jax.ShapeDtypeStruct((1, F), jnp.float32),
        ],
    )(a_prelu.reshape(1), A, XW, XWp)


def _readout_body(h_ref, hn_ref, cs_ref, wfc_ref, o1_ref, o2_ref):
    s = jax.nn.sigmoid(cs_ref[...] * (1.0 / N))
    x = jnp.sum(wfc_ref[...] * s, axis=1)
    o1_ref[...] = jnp.sum(h_ref[...] * x[None, :], axis=1)
    o2_ref[...] = jnp.sum(hn_ref[...] * x[None, :], axis=1)


def _readout(H, Hn, cs, Wfc):
    return pl.pallas_call(
        _readout_body,
        grid=(1,),
        in_specs=[
            pl.BlockSpec((N, F), lambda i: (0, 0)),
            pl.BlockSpec((N, F), lambda i: (0, 0)),
            pl.BlockSpec((1, F), lambda i: (0, 0)),
            pl.BlockSpec((F, F), lambda i: (0, 0)),
        ],
        out_specs=[
            pl.BlockSpec((N,), lambda i: (0,)),
            pl.BlockSpec((N,), lambda i: (0,)),
        ],
        out_shape=[
            jax.ShapeDtypeStruct((N,), jnp.float32),
            jax.ShapeDtypeStruct((N,), jnp.float32),
        ],
    )(H, Hn, cs, Wfc)


def kernel(X, A, W, a_prelu, Wfc, perm):
    XW = _matmul_xw(X, W)
    XWp = jnp.take(XW, perm, axis=0)
    H, Hn, cs = _main(A, XW, XWp, a_prelu)
    o1, o2 = _readout(H, Hn, cs, Wfc)
    out = jnp.concatenate([o1, o2], axis=0)
    labels = jnp.concatenate([
        jnp.ones((N,), dtype=jnp.float32),
        jnp.zeros((N,), dtype=jnp.float32),
    ])
    return (out, labels, jnp.array(0.0, dtype=jnp.float32))


# speedup vs baseline: 1.5513x; 1.1181x over previous
import jax
import jax.numpy as jnp
from jax.experimental import pallas as pl
from jax.experimental.pallas import tpu as pltpu

N = 10000
F = 128
TM = 400


def _xw_body(x_ref, w_ref, xw_ref):
    xw_ref[...] = jnp.dot(x_ref[...], w_ref[...],
                          preferred_element_type=jnp.float32
                          ).astype(jnp.bfloat16)


def _matmul_xw(X, W):
    return pl.pallas_call(
        _xw_body,
        grid=(5,),
        in_specs=[
            pl.BlockSpec((2000, F), lambda i: (i, 0)),
            pl.BlockSpec((F, F), lambda i: (0, 0)),
        ],
        out_specs=pl.BlockSpec((2000, F), lambda i: (i, 0)),
        out_shape=jax.ShapeDtypeStruct((N, F), jnp.bfloat16),
    )(X, W)


def _main_body(ap_ref, a_ref, xw_ref, xwp_ref, h_ref, hn_ref, cs_ref):
    a = a_ref[...].astype(jnp.bfloat16)
    c1 = jnp.dot(a, xw_ref[...], preferred_element_type=jnp.float32)
    c2 = jnp.dot(a, xwp_ref[...], preferred_element_type=jnp.float32)
    al = ap_ref[0]
    h1 = jnp.where(c1 >= 0, c1, al * c1)
    h2 = jnp.where(c2 >= 0, c2, al * c2)
    h_ref[...] = h1.astype(jnp.bfloat16)
    hn_ref[...] = h2.astype(jnp.bfloat16)
    part = jnp.sum(h1, axis=0, keepdims=True)

    @pl.when(pl.program_id(0) == 0)
    def _init():
        cs_ref[...] = part

    @pl.when(pl.program_id(0) > 0)
    def _acc():
        cs_ref[...] += part


def _main(A, XW, XWp, a_prelu):
    return pl.pallas_call(
        _main_body,
        grid=(N // TM,),
        in_specs=[
            pl.BlockSpec(memory_space=pltpu.SMEM),
            pl.BlockSpec((TM, N), lambda i: (i, 0)),
            pl.BlockSpec((N, F), lambda i: (0, 0)),
            pl.BlockSpec((N, F), lambda i: (0, 0)),
        ],
        out_specs=[
            pl.BlockSpec((TM, F), lambda i: (i, 0)),
            pl.BlockSpec((TM, F), lambda i: (i, 0)),
            pl.BlockSpec((1, F), lambda i: (0, 0)),
        ],
        out_shape=[
            jax.ShapeDtypeStruct((N, F), jnp.bfloat16),
            jax.ShapeDtypeStruct((N, F), jnp.bfloat16),
            jax.ShapeDtypeStruct((1, F), jnp.float32),
        ],
    )(a_prelu.reshape(1), A, XW, XWp)


def _readout_body(h_ref, hn_ref, cs_ref, wfc_ref, o1_ref, o2_ref):
    s = jax.nn.sigmoid(cs_ref[...] * (1.0 / N))
    x = jnp.sum(wfc_ref[...] * s, axis=1)
    o1_ref[...] = jnp.sum(h_ref[...] * x[None, :], axis=1)
    o2_ref[...] = jnp.sum(hn_ref[...] * x[None, :], axis=1)


def _readout(H, Hn, cs, Wfc):
    return pl.pallas_call(
        _readout_body,
        grid=(1,),
        in_specs=[
            pl.BlockSpec((N, F), lambda i: (0, 0)),
            pl.BlockSpec((N, F), lambda i: (0, 0)),
            pl.BlockSpec((1, F), lambda i: (0, 0)),
            pl.BlockSpec((F, F), lambda i: (0, 0)),
        ],
        out_specs=[
            pl.BlockSpec((N,), lambda i: (0,)),
            pl.BlockSpec((N,), lambda i: (0,)),
        ],
        out_shape=[
            jax.ShapeDtypeStruct((N,), jnp.float32),
            jax.ShapeDtypeStruct((N,), jnp.float32),
        ],
    )(H, Hn, cs, Wfc)


def kernel(X, A, W, a_prelu, Wfc, perm):
    XW = _matmul_xw(X, W)
    XWp = jnp.take(XW, perm, axis=0)
    H, Hn, cs = _main(A, XW, XWp, a_prelu)
    o1, o2 = _readout(H, Hn, cs, Wfc)
    out = jnp.concatenate([o1, o2], axis=0)
    labels = jnp.concatenate([
        jnp.ones((N,), dtype=jnp.float32),
        jnp.zeros((N,), dtype=jnp.float32),
    ])
    return (out, labels, jnp.array(0.0, dtype=jnp.float32))
